# baseline (speedup 1.0000x reference)
import functools

import jax
import jax.numpy as jnp
from jax import lax
from jax.experimental import pallas as pl
from jax.experimental.pallas import tpu as pltpu
from jax.experimental.pallas import tpu_sc as plsc

_K = 9
_KPAD = 16


def _erf(x):
    a1, a2, a3, a4, a5 = 0.254829592, -0.284496736, 1.421413741, -1.453152027, 1.061405429
    p = 0.3275911
    ax = jnp.abs(x)
    t = 1.0 / (1.0 + p * ax)
    poly = ((((a5 * t + a4) * t + a3) * t + a2) * t + a1) * t
    y = 1.0 - poly * jnp.exp(-ax * ax)
    return jnp.sign(x) * y


def _p1(xT_ref, w1t_ref, b1_ref, y_ref, s_ref, ss_ref):
    xT = xT_ref[0]
    y = jnp.dot(xT, w1t_ref[...], preferred_element_type=jnp.float32) + b1_ref[...]
    y_ref[0] = y
    s_ref[0, 0] = jnp.sum(y, axis=0)
    ss_ref[0, 0] = jnp.sum(y * y, axis=0)


def _p2(y_ref, sc_ref, sh_ref, wdT_ref, wjT_ref, bgc_ref, idx_ref, aT_ref, bfT_ref, *, TN):
    b = pl.program_id(0)
    t = pl.program_id(1)
    N = y_ref.shape[1]
    y = y_ref[0]
    h = y * sc_ref[...] + sh_ref[...]
    nrm = jnp.sqrt(jnp.sum(h * h, axis=1, keepdims=True))
    xn = h / (nrm + 1e-12)
    x2 = jnp.sum(xn * xn, axis=1, keepdims=True)

    h_t = y_ref[0, pl.ds(t * TN, TN), :] * sc_ref[...] + sh_ref[...]
    nrm_t = jnp.sqrt(jnp.sum(h_t * h_t, axis=1, keepdims=True))
    xn_t = h_t / (nrm_t + 1e-12)

    Laug = jnp.concatenate([xn_t, jnp.ones((TN, 1), jnp.float32)], axis=1)
    Raug = jnp.concatenate([2.0 * xn, -x2], axis=1)
    score = lax.dot_general(Laug, Raug, (((1,), (1,)), ((), ())),
                            preferred_element_type=jnp.float32)

    cols = lax.broadcasted_iota(jnp.int32, (TN, N), 1)
    lanek = lax.broadcasted_iota(jnp.int32, (TN, _KPAD), 1)
    rowid = t * TN + lax.broadcasted_iota(jnp.int32, (TN, 1), 0)
    acc = jnp.where(lanek == 0, rowid, 0)
    vals = jnp.where(cols == rowid, -jnp.inf, score)
    for k in range(1, _K):
        rmax = jnp.max(vals, axis=1, keepdims=True)
        am = jnp.min(jnp.where(vals == rmax, cols, jnp.int32(N)), axis=1)
        acc = jnp.where(lanek == k, am[:, None], acc)
        vals = jnp.where(cols == am[:, None], -jnp.inf, vals)
    idx_ref[0] = acc + b * N
    aT_ref[0] = lax.dot_general(h_t, wdT_ref[...], (((1,), (0,)), ((), ())),
                                preferred_element_type=jnp.float32) + bgc_ref[...]
    bfT_ref[0] = lax.dot_general(h_t, wjT_ref[...], (((1,), (0,)), ((), ())),
                                 preferred_element_type=jnp.float32)


def _sc_gather_agg(bf_flat, idx_flat, a_flat):
    R, DP = bf_flat.shape
    D = a_flat.shape[1]
    KG = _K - 1
    info = plsc.get_sparse_core_info()
    NW = info.num_cores * info.num_subcores
    L = info.num_lanes
    NL = D // L
    rows_per_w = R // NW
    CH = 8
    CHK = CH * KG
    steps = rows_per_w // CH
    mesh = plsc.VectorSubcoreMesh(core_axis_name="c", subcore_axis_name="s")

    @functools.partial(
        pl.kernel,
        mesh=mesh,
        out_type=[
            jax.ShapeDtypeStruct((R, D), jnp.float32),
            jax.ShapeDtypeStruct((NW, D), jnp.float32),
            jax.ShapeDtypeStruct((NW, D), jnp.float32),
        ],
    scratch_types=[
            pltpu.VMEM((rows_per_w * KG,), jnp.int32),
            pltpu.VMEM((2, CHK, DP), jnp.float32),
            pltpu.VMEM((2, CH, D), jnp.float32),
            pltpu.VMEM((2, CH, DP), jnp.float32),
            pltpu.VMEM((2, CH, D), jnp.float32),
            pltpu.VMEM((D,), jnp.float32),
            pltpu.VMEM((D,), jnp.float32),
            pltpu.SemaphoreType.DMA,
            pltpu.SemaphoreType.DMA,
            pltpu.SemaphoreType.DMA,
            pltpu.SemaphoreType.DMA,
            pltpu.SemaphoreType.DMA,
            pltpu.SemaphoreType.DMA,
        ],
    )
    def k(bf_hbm, idx_hbm, a_hbm, agg_hbm, s_hbm, ss_hbm,
          idx_v, rows_v, a_v, self_v, out_v, s_v, ss_v,
          sg0, sg1, sa0, sa1, so0, so1):
        wid = lax.axis_index("s") * info.num_cores + lax.axis_index("c")
        base = wid * rows_per_w
        sg = (sg0, sg1)
        sa = (sa0, sa1)
        so = (so0, so1)
        pltpu.sync_copy(idx_hbm.at[pl.ds(base * KG, rows_per_w * KG)], idx_v)
        zero = jnp.zeros((L,), jnp.float32)
        for c in range(NL):
            s_v[pl.ds(c * L, L)] = zero
            ss_v[pl.ds(c * L, L)] = zero

        def launch(s, slot):
            pltpu.make_async_copy(
                bf_hbm.at[idx_v.at[pl.ds(s * CHK, CHK)]], rows_v.at[slot], sg[slot]
            ).start()
            pltpu.make_async_copy(
                a_hbm.at[pl.ds(base + s * CH, CH)], a_v.at[slot], sa[slot]
            ).start()
            pltpu.make_async_copy(
                bf_hbm.at[pl.ds(base + s * CH, CH)], self_v.at[slot], sa[slot]
            ).start()

        def wait_launch(s, slot):
            pltpu.make_async_copy(
                bf_hbm.at[idx_v.at[pl.ds(s * CHK, CHK)]], rows_v.at[slot], sg[slot]
            ).wait()
            pltpu.make_async_copy(
                a_hbm.at[pl.ds(base + s * CH, CH)], a_v.at[slot], sa[slot]
            ).wait()
            pltpu.make_async_copy(
                bf_hbm.at[pl.ds(base + s * CH, CH)], self_v.at[slot], sa[slot]
            ).wait()

        launch(0, 0)

        def pair(gp, carry):
            for b in (0, 1):
                s = 2 * gp + b

                @pl.when(s + 1 < steps)
                def _():
                    launch(s + 1, 1 - b)

                wait_launch(s, b)

                @pl.when(s >= 2)
                def _():
                    pltpu.make_async_copy(
                        out_v.at[b], agg_hbm.at[pl.ds(base + (s - 2) * CH, CH)], so[b]
                    ).wait()

                for i in range(CH):
                    for c in range(NL):
                        sl = pl.ds(c * L, L)
                        acc = self_v[b, i, sl]
                        for kk in range(KG):
                            acc = jnp.maximum(acc, rows_v[b, i * KG + kk, sl])
                        agg = acc + a_v[b, i, sl]
                        out_v[b, i, sl] = agg
                        s_v[sl] = s_v[sl] + agg
                        ss_v[sl] = ss_v[sl] + agg * agg

                pltpu.make_async_copy(
                    out_v.at[b], agg_hbm.at[pl.ds(base + s * CH, CH)], so[b]
                ).start()
            return carry

        lax.fori_loop(0, steps // 2, pair, 0)
        for b in (0, 1):
            pltpu.make_async_copy(
                out_v.at[b], agg_hbm.at[pl.ds(base + (steps - 2 + b) * CH, CH)], so[b]
            ).wait()
        pltpu.sync_copy(s_v, s_hbm.at[wid])
        pltpu.sync_copy(ss_v, ss_hbm.at[wid])

    return k(bf_flat, idx_flat, a_flat)


def _p5(agg_ref, sc_ref, sh_ref, w2t_ref, b2_ref, outp_ref, s_ref, ss_ref):
    z = agg_ref[0] * sc_ref[...] + sh_ref[...]
    gx = 0.5 * z * (1.0 + _erf(z * 0.7071067811865476))
    o = jnp.dot(gx, w2t_ref[...], preferred_element_type=jnp.float32) + b2_ref[...]
    outp_ref[0] = o
    s_ref[0, 0] = jnp.sum(o, axis=0)
    ss_ref[0, 0] = jnp.sum(o * o, axis=0)


def _p6(outp_ref, sc_ref, sh_ref, xT_ref, out_ref):
    out_ref[0] = outp_ref[0] * sc_ref[...] + sh_ref[...] + xT_ref[0]


def _affine(s_p, ss_p, g, be, cnt):
    m = jnp.sum(s_p, axis=(0, 1)) / cnt
    v = jnp.sum(ss_p, axis=(0, 1)) / cnt - m * m
    sc = g / jnp.sqrt(v + 1e-5)
    sh = be - m * sc
    return sc[None, :], sh[None, :]


def kernel(x, W_fc1, b_fc1, g1, be1, W_gc, b_gc, g2, be2, W_fc2, b_fc2, g3, be3):
    B, C, H, Wd = x.shape
    N = H * Wd
    HID = W_gc.shape[0]
    TN = 256 if N % 256 == 0 else N
    NT = N // TN
    f32 = jnp.float32

    xT = jnp.transpose(x.reshape(B, C, N), (0, 2, 1))
    w1t = W_fc1.T
    wi, wj = W_gc[:, :C], W_gc[:, C:]
    wdT = (wi - wj).T
    HIDP = ((HID + 127) // 128) * 128
    wjT = jnp.pad(wj.T, ((0, 0), (0, HIDP - HID)))
    w2t = W_fc2.T

    yT, s1p, ss1p = pl.pallas_call(
        _p1,
        grid=(B,),
        in_specs=[
            pl.BlockSpec((1, N, C), lambda b: (b, 0, 0)),
            pl.BlockSpec((C, C), lambda b: (0, 0)),
            pl.BlockSpec((1, C), lambda b: (0, 0)),
        ],
        out_specs=[
            pl.BlockSpec((1, N, C), lambda b: (b, 0, 0)),
            pl.BlockSpec((1, 1, C), lambda b: (b, 0, 0)),
            pl.BlockSpec((1, 1, C), lambda b: (b, 0, 0)),
        ],
        out_shape=[
            jax.ShapeDtypeStruct((B, N, C), f32),
            jax.ShapeDtypeStruct((B, 1, C), f32),
            jax.ShapeDtypeStruct((B, 1, C), f32),
        ],
    )(xT, w1t, b_fc1[None, :])
    sc1, sh1 = _affine(s1p, ss1p, g1, be1, B * N)

    NSPLIT = 4 if B % 4 == 0 else 1
    BH = B // NSPLIT
    agg_chunks, st2 = [], []
    for h in range(NSPLIT):
        idx_h, aT_h, bfT_h = pl.pallas_call(
            functools.partial(_p2, TN=TN),
            grid=(BH, NT),
            in_specs=[
                pl.BlockSpec((1, N, C), lambda b, t, h=h: (h * BH + b, 0, 0)),
                pl.BlockSpec((1, C), lambda b, t: (0, 0)),
                pl.BlockSpec((1, C), lambda b, t: (0, 0)),
                pl.BlockSpec((C, HID), lambda b, t: (0, 0)),
                pl.BlockSpec((C, HIDP), lambda b, t: (0, 0)),
                pl.BlockSpec((1, HID), lambda b, t: (0, 0)),
            ],
            out_specs=[
                pl.BlockSpec((1, TN, _KPAD), lambda b, t: (b, t, 0)),
                pl.BlockSpec((1, TN, HID), lambda b, t: (b, t, 0)),
                pl.BlockSpec((1, TN, HIDP), lambda b, t: (b, t, 0)),
            ],
            out_shape=[
                jax.ShapeDtypeStruct((BH, N, _KPAD), jnp.int32),
                jax.ShapeDtypeStruct((BH, N, HID), f32),
                jax.ShapeDtypeStruct((BH, N, HIDP), f32),
            ],
        )(yT, sc1, sh1, wdT, wjT, b_gc[None, :])

        idx_flat = idx_h[:, :, 1:_K].reshape(BH * N * (_K - 1))
        agg_h, s2p_h, ss2p_h = _sc_gather_agg(
            bfT_h.reshape(BH * N, HIDP), idx_flat, aT_h.reshape(BH * N, HID))
        agg_chunks.append(agg_h.reshape(BH, N, HID))
        st2.append((s2p_h, ss2p_h))

    agg = jnp.concatenate(agg_chunks, axis=0) if NSPLIT > 1 else agg_chunks[0]
    s2p = jnp.concatenate([s for s, _ in st2], axis=0)
    ss2p = jnp.concatenate([ss for _, ss in st2], axis=0)
    sc2, sh2 = _affine(s2p[:, None, :], ss2p[:, None, :], g2, be2, B * N)

    outp, s3p, ss3p = pl.pallas_call(
        _p5,
        grid=(B, NT),
        in_specs=[
            pl.BlockSpec((1, TN, HID), lambda b, t: (b, t, 0)),
            pl.BlockSpec((1, HID), lambda b, t: (0, 0)),
            pl.BlockSpec((1, HID), lambda b, t: (0, 0)),
            pl.BlockSpec((HID, C), lambda b, t: (0, 0)),
            pl.BlockSpec((1, C), lambda b, t: (0, 0)),
        ],
        out_specs=[
            pl.BlockSpec((1, TN, C), lambda b, t: (b, t, 0)),
            pl.BlockSpec((1, 1, C), lambda b, t: (b * NT + t, 0, 0)),
            pl.BlockSpec((1, 1, C), lambda b, t: (b * NT + t, 0, 0)),
        ],
        out_shape=[
            jax.ShapeDtypeStruct((B, N, C), f32),
            jax.ShapeDtypeStruct((B * NT, 1, C), f32),
            jax.ShapeDtypeStruct((B * NT, 1, C), f32),
        ],
    )(agg, sc2, sh2, w2t, b_fc2[None, :])
    sc3, sh3 = _affine(s3p, ss3p, g3, be3, B * N)

    outT = pl.pallas_call(
        _p6,
        grid=(B,),
        in_specs=[
            pl.BlockSpec((1, N, C), lambda b: (b, 0, 0)),
            pl.BlockSpec((1, C), lambda b: (0, 0)),
            pl.BlockSpec((1, C), lambda b: (0, 0)),
            pl.BlockSpec((1, N, C), lambda b: (b, 0, 0)),
        ],
        out_specs=pl.BlockSpec((1, N, C), lambda b: (b, 0, 0)),
        out_shape=jax.ShapeDtypeStruct((B, N, C), f32),
    )(outp, sc3, sh3, xT)

    return jnp.transpose(outT, (0, 2, 1)).reshape(B, C, H, Wd)


# speedup vs baseline: 1.0644x; 1.0644x over previous
import functools

import jax
import jax.numpy as jnp
from jax import lax
from jax.experimental import pallas as pl
from jax.experimental.pallas import tpu as pltpu
from jax.experimental.pallas import tpu_sc as plsc

_K = 9
_KPAD = 16


def _erf(x):
    a1, a2, a3, a4, a5 = 0.254829592, -0.284496736, 1.421413741, -1.453152027, 1.061405429
    p = 0.3275911
    ax = jnp.abs(x)
    t = 1.0 / (1.0 + p * ax)
    poly = ((((a5 * t + a4) * t + a3) * t + a2) * t + a1) * t
    y = 1.0 - poly * jnp.exp(-ax * ax)
    return jnp.sign(x) * y


def _p1(xT_ref, w1t_ref, b1_ref, y_ref, s_ref, ss_ref):
    xT = xT_ref[0]
    y = jnp.dot(xT, w1t_ref[...], preferred_element_type=jnp.float32) + b1_ref[...]
    y_ref[0] = y
    s_ref[0, 0] = jnp.sum(y, axis=0)
    ss_ref[0, 0] = jnp.sum(y * y, axis=0)


def _p2(y_ref, sc_ref, sh_ref, wdT_ref, wjT_ref, bgc_ref, idx_ref, aT_ref,
        bfT_ref, h_sc, laug_sc, *, TN):
    b = pl.program_id(0)
    N = y_ref.shape[1]
    NT = N // TN
    y = y_ref[0]
    h = y * sc_ref[...] + sh_ref[...]
    h_sc[...] = h
    nrm = jnp.sqrt(jnp.sum(h * h, axis=1, keepdims=True))
    xn = h / (nrm + 1e-12)
    x2 = jnp.sum(xn * xn, axis=1, keepdims=True)
    laug_sc[...] = jnp.concatenate([xn, jnp.ones((N, 1), jnp.float32)], axis=1)
    raug = jnp.concatenate([2.0 * xn, -x2], axis=1)

    cols = lax.broadcasted_iota(jnp.int32, (TN, N), 1)
    negcols = -cols.astype(jnp.float32)
    lanek = lax.broadcasted_iota(jnp.int32, (TN, _KPAD), 1)
    iota0 = lax.broadcasted_iota(jnp.int32, (TN, 1), 0)

    def tile(t, carry):
        lt = laug_sc[pl.ds(t * TN, TN), :]
        score = lax.dot_general(lt, raug, (((1,), (1,)), ((), ())),
                                preferred_element_type=jnp.float32)
        rowid = t * TN + iota0
        acc = jnp.where(lanek == 0, rowid, 0)
        vals = jnp.where(cols == rowid, -jnp.inf, score)
        for k in range(1, _K):
            rmax = jnp.max(vals, axis=1, keepdims=True)
            amf = jnp.max(jnp.where(vals == rmax, negcols, -jnp.inf), axis=1)
            am = (-amf).astype(jnp.int32)
            acc = jnp.where(lanek == k, am[:, None], acc)
            vals = jnp.where(cols == am[:, None], -jnp.inf, vals)
        idx_ref[0, pl.ds(t * TN, TN), :] = acc + b * N
        ht = h_sc[pl.ds(t * TN, TN), :]
        aT_ref[0, pl.ds(t * TN, TN), :] = lax.dot_general(
            ht, wdT_ref[...], (((1,), (0,)), ((), ())),
            preferred_element_type=jnp.float32) + bgc_ref[...]
        bfT_ref[0, pl.ds(t * TN, TN), :] = lax.dot_general(
            ht, wjT_ref[...], (((1,), (0,)), ((), ())),
            preferred_element_type=jnp.float32)
        return carry

    lax.fori_loop(0, NT, tile, 0)


def _sc_gather_agg(bf_flat, idx_flat, a_flat):
    R, DP = bf_flat.shape
    D = a_flat.shape[1]
    KG = _K
    info = plsc.get_sparse_core_info()
    NW = info.num_cores * info.num_subcores
    L = info.num_lanes
    NL = D // L
    rows_per_w = R // NW
    CH = 8
    CHK = CH * KG
    steps = rows_per_w // CH
    mesh = plsc.VectorSubcoreMesh(core_axis_name="c", subcore_axis_name="s")

    @functools.partial(
        pl.kernel,
        mesh=mesh,
        out_type=[
            jax.ShapeDtypeStruct((R, D), jnp.float32),
            jax.ShapeDtypeStruct((NW, D), jnp.float32),
            jax.ShapeDtypeStruct((NW, D), jnp.float32),
        ],
    scratch_types=[
            pltpu.VMEM((rows_per_w * KG,), jnp.int32),
            pltpu.VMEM((2, CHK, DP), jnp.float32),
            pltpu.VMEM((2, CH, D), jnp.float32),
            pltpu.VMEM((2, CH, D), jnp.float32),
            pltpu.VMEM((D,), jnp.float32),
            pltpu.VMEM((D,), jnp.float32),
            pltpu.SemaphoreType.DMA,
            pltpu.SemaphoreType.DMA,
            pltpu.SemaphoreType.DMA,
            pltpu.SemaphoreType.DMA,
            pltpu.SemaphoreType.DMA,
            pltpu.SemaphoreType.DMA,
        ],
    )
    def k(bf_hbm, idx_hbm, a_hbm, agg_hbm, s_hbm, ss_hbm,
          idx_v, rows_v, a_v, out_v, s_v, ss_v,
          sg0, sg1, sa0, sa1, so0, so1):
        wid = lax.axis_index("s") * info.num_cores + lax.axis_index("c")
        base = wid * rows_per_w
        sg = (sg0, sg1)
        sa = (sa0, sa1)
        so = (so0, so1)
        pltpu.sync_copy(idx_hbm.at[pl.ds(base * KG, rows_per_w * KG)], idx_v)
        zero = jnp.zeros((L,), jnp.float32)
        for c in range(NL):
            s_v[pl.ds(c * L, L)] = zero
            ss_v[pl.ds(c * L, L)] = zero

        def launch(s, slot):
            pltpu.make_async_copy(
                bf_hbm.at[idx_v.at[pl.ds(s * CHK, CHK)]], rows_v.at[slot], sg[slot]
            ).start()
            pltpu.make_async_copy(
                a_hbm.at[pl.ds(base + s * CH, CH)], a_v.at[slot], sa[slot]
            ).start()

        def wait_launch(s, slot):
            pltpu.make_async_copy(
                bf_hbm.at[idx_v.at[pl.ds(s * CHK, CHK)]], rows_v.at[slot], sg[slot]
            ).wait()
            pltpu.make_async_copy(
                a_hbm.at[pl.ds(base + s * CH, CH)], a_v.at[slot], sa[slot]
            ).wait()

        launch(0, 0)

        def pair(gp, carry):
            for b in (0, 1):
                s = 2 * gp + b

                @pl.when(s + 1 < steps)
                def _():
                    launch(s + 1, 1 - b)

                wait_launch(s, b)

                @pl.when(s >= 2)
                def _():
                    pltpu.make_async_copy(
                        out_v.at[b], agg_hbm.at[pl.ds(base + (s - 2) * CH, CH)], so[b]
                    ).wait()

                for i in range(CH):
                    for c in range(NL):
                        sl = pl.ds(c * L, L)
                        acc = rows_v[b, i * KG, sl]
                        for kk in range(1, KG):
                            acc = jnp.maximum(acc, rows_v[b, i * KG + kk, sl])
                        agg = acc + a_v[b, i, sl]
                        out_v[b, i, sl] = agg
                        s_v[sl] = s_v[sl] + agg
                        ss_v[sl] = ss_v[sl] + agg * agg

                pltpu.make_async_copy(
                    out_v.at[b], agg_hbm.at[pl.ds(base + s * CH, CH)], so[b]
                ).start()
            return carry

        lax.fori_loop(0, steps // 2, pair, 0)
        for b in (0, 1):
            pltpu.make_async_copy(
                out_v.at[b], agg_hbm.at[pl.ds(base + (steps - 2 + b) * CH, CH)], so[b]
            ).wait()
        pltpu.sync_copy(s_v, s_hbm.at[wid])
        pltpu.sync_copy(ss_v, ss_hbm.at[wid])

    return k(bf_flat, idx_flat, a_flat)


def _p5(agg_ref, sc_ref, sh_ref, w2t_ref, b2_ref, outp_ref, s_ref, ss_ref):
    z = agg_ref[0] * sc_ref[...] + sh_ref[...]
    gx = 0.5 * z * (1.0 + _erf(z * 0.7071067811865476))
    o = jnp.dot(gx, w2t_ref[...], preferred_element_type=jnp.float32) + b2_ref[...]
    outp_ref[0] = o
    s_ref[0, 0] = jnp.sum(o, axis=0)
    ss_ref[0, 0] = jnp.sum(o * o, axis=0)


def _p6(outp_ref, sc_ref, sh_ref, xT_ref, out_ref):
    out_ref[0] = outp_ref[0] * sc_ref[...] + sh_ref[...] + xT_ref[0]


def _affine(s_p, ss_p, g, be, cnt):
    m = jnp.sum(s_p, axis=(0, 1)) / cnt
    v = jnp.sum(ss_p, axis=(0, 1)) / cnt - m * m
    sc = g / jnp.sqrt(v + 1e-5)
    sh = be - m * sc
    return sc[None, :], sh[None, :]


def kernel(x, W_fc1, b_fc1, g1, be1, W_gc, b_gc, g2, be2, W_fc2, b_fc2, g3, be3):
    B, C, H, Wd = x.shape
    N = H * Wd
    HID = W_gc.shape[0]
    TN = 256 if N % 256 == 0 else N
    NT = N // TN
    f32 = jnp.float32

    xT = jnp.transpose(x.reshape(B, C, N), (0, 2, 1))
    w1t = W_fc1.T
    wi, wj = W_gc[:, :C], W_gc[:, C:]
    wdT = (wi - wj).T
    HIDP = ((HID + 127) // 128) * 128
    wjT = jnp.pad(wj.T, ((0, 0), (0, HIDP - HID)))
    w2t = W_fc2.T

    yT, s1p, ss1p = pl.pallas_call(
        _p1,
        grid=(B,),
        in_specs=[
            pl.BlockSpec((1, N, C), lambda b: (b, 0, 0)),
            pl.BlockSpec((C, C), lambda b: (0, 0)),
            pl.BlockSpec((1, C), lambda b: (0, 0)),
        ],
        out_specs=[
            pl.BlockSpec((1, N, C), lambda b: (b, 0, 0)),
            pl.BlockSpec((1, 1, C), lambda b: (b, 0, 0)),
            pl.BlockSpec((1, 1, C), lambda b: (b, 0, 0)),
        ],
        out_shape=[
            jax.ShapeDtypeStruct((B, N, C), f32),
            jax.ShapeDtypeStruct((B, 1, C), f32),
            jax.ShapeDtypeStruct((B, 1, C), f32),
        ],
    )(xT, w1t, b_fc1[None, :])
    sc1, sh1 = _affine(s1p, ss1p, g1, be1, B * N)

    NSPLIT = 4 if B % 4 == 0 else 1
    BH = B // NSPLIT
    agg_chunks, st2 = [], []
    for h in range(NSPLIT):
        idx_h, aT_h, bfT_h = pl.pallas_call(
            functools.partial(_p2, TN=TN),
            grid=(BH,),
            in_specs=[
                pl.BlockSpec((1, N, C), lambda b, h=h: (h * BH + b, 0, 0)),
                pl.BlockSpec((1, C), lambda b: (0, 0)),
                pl.BlockSpec((1, C), lambda b: (0, 0)),
                pl.BlockSpec((C, HID), lambda b: (0, 0)),
                pl.BlockSpec((C, HIDP), lambda b: (0, 0)),
                pl.BlockSpec((1, HID), lambda b: (0, 0)),
            ],
            out_specs=[
                pl.BlockSpec((1, N, _KPAD), lambda b: (b, 0, 0)),
                pl.BlockSpec((1, N, HID), lambda b: (b, 0, 0)),
                pl.BlockSpec((1, N, HIDP), lambda b: (b, 0, 0)),
            ],
            out_shape=[
                jax.ShapeDtypeStruct((BH, N, _KPAD), jnp.int32),
                jax.ShapeDtypeStruct((BH, N, HID), f32),
                jax.ShapeDtypeStruct((BH, N, HIDP), f32),
            ],
            scratch_shapes=[
                pltpu.VMEM((N, C), f32),
                pltpu.VMEM((N, C + 1), f32),
            ],
        )(yT, sc1, sh1, wdT, wjT, b_gc[None, :])

        idx_flat = idx_h[:, :, :_K].reshape(BH * N * _K)
        agg_h, s2p_h, ss2p_h = _sc_gather_agg(
            bfT_h.reshape(BH * N, HIDP), idx_flat, aT_h.reshape(BH * N, HID))
        agg_chunks.append(agg_h.reshape(BH, N, HID))
        st2.append((s2p_h, ss2p_h))

    agg = jnp.concatenate(agg_chunks, axis=0) if NSPLIT > 1 else agg_chunks[0]
    s2p = jnp.concatenate([s for s, _ in st2], axis=0)
    ss2p = jnp.concatenate([ss for _, ss in st2], axis=0)
    sc2, sh2 = _affine(s2p[:, None, :], ss2p[:, None, :], g2, be2, B * N)

    outp, s3p, ss3p = pl.pallas_call(
        _p5,
        grid=(B, NT),
        in_specs=[
            pl.BlockSpec((1, TN, HID), lambda b, t: (b, t, 0)),
            pl.BlockSpec((1, HID), lambda b, t: (0, 0)),
            pl.BlockSpec((1, HID), lambda b, t: (0, 0)),
            pl.BlockSpec((HID, C), lambda b, t: (0, 0)),
            pl.BlockSpec((1, C), lambda b, t: (0, 0)),
        ],
        out_specs=[
            pl.BlockSpec((1, TN, C), lambda b, t: (b, t, 0)),
            pl.BlockSpec((1, 1, C), lambda b, t: (b * NT + t, 0, 0)),
            pl.BlockSpec((1, 1, C), lambda b, t: (b * NT + t, 0, 0)),
        ],
        out_shape=[
            jax.ShapeDtypeStruct((B, N, C), f32),
            jax.ShapeDtypeStruct((B * NT, 1, C), f32),
            jax.ShapeDtypeStruct((B * NT, 1, C), f32),
        ],
    )(agg, sc2, sh2, w2t, b_fc2[None, :])
    sc3, sh3 = _affine(s3p, ss3p, g3, be3, B * N)

    outT = pl.pallas_call(
        _p6,
        grid=(B,),
        in_specs=[
            pl.BlockSpec((1, N, C), lambda b: (b, 0, 0)),
            pl.BlockSpec((1, C), lambda b: (0, 0)),
            pl.BlockSpec((1, C), lambda b: (0, 0)),
            pl.BlockSpec((1, N, C), lambda b: (b, 0, 0)),
        ],
        out_specs=pl.BlockSpec((1, N, C), lambda b: (b, 0, 0)),
        out_shape=jax.ShapeDtypeStruct((B, N, C), f32),
    )(outp, sc3, sh3, xT)

    return jnp.transpose(outT, (0, 2, 1)).reshape(B, C, H, Wd)


# speedup vs baseline: 1.0659x; 1.0014x over previous
import functools

import jax
import jax.numpy as jnp
from jax import lax
from jax.experimental import pallas as pl
from jax.experimental.pallas import tpu as pltpu
from jax.experimental.pallas import tpu_sc as plsc

_K = 9
_KPAD = 16


def _erf(x):
    a1, a2, a3, a4, a5 = 0.254829592, -0.284496736, 1.421413741, -1.453152027, 1.061405429
    p = 0.3275911
    ax = jnp.abs(x)
    t = 1.0 / (1.0 + p * ax)
    poly = ((((a5 * t + a4) * t + a3) * t + a2) * t + a1) * t
    y = 1.0 - poly * jnp.exp(-ax * ax)
    return jnp.sign(x) * y


def _p1(xT_ref, w1t_ref, b1_ref, y_ref, s_ref, ss_ref):
    xT = xT_ref[0]
    y = jnp.dot(xT, w1t_ref[...], preferred_element_type=jnp.float32) + b1_ref[...]
    y_ref[0] = y
    s_ref[0, 0] = jnp.sum(y, axis=0)
    ss_ref[0, 0] = jnp.sum(y * y, axis=0)


def _p2(y_ref, sc_ref, sh_ref, wdT_ref, wjT_ref, bgc_ref, idx_ref, aT_ref,
        bfT_ref, h_sc, laug_sc, *, TN):
    b = pl.program_id(0)
    N = y_ref.shape[1]
    NT = N // TN
    y = y_ref[0]
    h = y * sc_ref[...] + sh_ref[...]
    h_sc[...] = h
    nrm = jnp.sqrt(jnp.sum(h * h, axis=1, keepdims=True))
    xn = h / (nrm + 1e-12)
    x2 = jnp.sum(xn * xn, axis=1, keepdims=True)
    laug_sc[...] = jnp.concatenate([xn, jnp.ones((N, 1), jnp.float32)], axis=1)
    raug = jnp.concatenate([2.0 * xn, -x2], axis=1)

    cols = lax.broadcasted_iota(jnp.int32, (TN, N), 1)
    negcols = -cols.astype(jnp.float32)
    lanek = lax.broadcasted_iota(jnp.int32, (TN, _KPAD), 1)
    iota0 = lax.broadcasted_iota(jnp.int32, (TN, 1), 0)

    def tile(t, carry):
        lt = laug_sc[pl.ds(t * TN, TN), :]
        score = lax.dot_general(lt, raug, (((1,), (1,)), ((), ())),
                                preferred_element_type=jnp.float32)
        rowid = t * TN + iota0
        acc = jnp.where(lanek == 0, rowid, 0)
        vals = jnp.where(cols == rowid, -jnp.inf, score)
        for k in range(1, _K):
            rmax = jnp.max(vals, axis=1, keepdims=True)
            amf = jnp.max(jnp.where(vals == rmax, negcols, -jnp.inf), axis=1)
            am = (-amf).astype(jnp.int32)
            acc = jnp.where(lanek == k, am[:, None], acc)
            vals = jnp.where(cols == am[:, None], -jnp.inf, vals)
        idx_ref[0, pl.ds(t * TN, TN), :] = acc + b * N
        ht = h_sc[pl.ds(t * TN, TN), :]
        aT_ref[0, pl.ds(t * TN, TN), :] = lax.dot_general(
            ht, wdT_ref[...], (((1,), (0,)), ((), ())),
            preferred_element_type=jnp.float32) + bgc_ref[...]
        bfT_ref[0, pl.ds(t * TN, TN), :] = lax.dot_general(
            ht, wjT_ref[...], (((1,), (0,)), ((), ())),
            preferred_element_type=jnp.float32)
        return carry

    lax.fori_loop(0, NT, tile, 0)


def _sc_gather_agg(bf_flat, idx_flat, a_flat):
    R, DP = bf_flat.shape
    D = a_flat.shape[1]
    KG = _K
    info = plsc.get_sparse_core_info()
    NW = info.num_cores * info.num_subcores
    L = info.num_lanes
    NL = D // L
    rows_per_w = R // NW
    CH = 8
    CHK = CH * KG
    steps = rows_per_w // CH
    mesh = plsc.VectorSubcoreMesh(core_axis_name="c", subcore_axis_name="s")

    @functools.partial(
        pl.kernel,
        mesh=mesh,
        compiler_params=pltpu.CompilerParams(use_tc_tiling_on_sc=False),
        out_type=[
            jax.ShapeDtypeStruct((R, D), jnp.float32),
            jax.ShapeDtypeStruct((NW, D), jnp.float32),
            jax.ShapeDtypeStruct((NW, D), jnp.float32),
        ],
    scratch_types=[
            pltpu.VMEM((rows_per_w * KG,), jnp.int32),
            pltpu.VMEM((2, CHK, DP), jnp.float32),
            pltpu.VMEM((2, CH, D), jnp.float32),
            pltpu.VMEM((2, CH, D), jnp.float32),
            pltpu.VMEM((D,), jnp.float32),
            pltpu.VMEM((D,), jnp.float32),
            pltpu.SemaphoreType.DMA,
            pltpu.SemaphoreType.DMA,
            pltpu.SemaphoreType.DMA,
            pltpu.SemaphoreType.DMA,
            pltpu.SemaphoreType.DMA,
            pltpu.SemaphoreType.DMA,
        ],
    )
    def k(bf_hbm, idx_hbm, a_hbm, agg_hbm, s_hbm, ss_hbm,
          idx_v, rows_v, a_v, out_v, s_v, ss_v,
          sg0, sg1, sa0, sa1, so0, so1):
        wid = lax.axis_index("s") * info.num_cores + lax.axis_index("c")
        base = wid * rows_per_w
        sg = (sg0, sg1)
        sa = (sa0, sa1)
        so = (so0, so1)
        pltpu.sync_copy(idx_hbm.at[pl.ds(base * KG, rows_per_w * KG)], idx_v)
        zero = jnp.zeros((L,), jnp.float32)
        for c in range(NL):
            s_v[pl.ds(c * L, L)] = zero
            ss_v[pl.ds(c * L, L)] = zero

        def launch(s, slot):
            pltpu.make_async_copy(
                bf_hbm.at[idx_v.at[pl.ds(s * CHK, CHK)]], rows_v.at[slot], sg[slot]
            ).start()
            pltpu.make_async_copy(
                a_hbm.at[pl.ds(base + s * CH, CH)], a_v.at[slot], sa[slot]
            ).start()

        def wait_launch(s, slot):
            pltpu.make_async_copy(
                bf_hbm.at[idx_v.at[pl.ds(s * CHK, CHK)]], rows_v.at[slot], sg[slot]
            ).wait()
            pltpu.make_async_copy(
                a_hbm.at[pl.ds(base + s * CH, CH)], a_v.at[slot], sa[slot]
            ).wait()

        launch(0, 0)

        def pair(gp, carry):
            for b in (0, 1):
                s = 2 * gp + b

                @pl.when(s + 1 < steps)
                def _():
                    launch(s + 1, 1 - b)

                wait_launch(s, b)

                @pl.when(s >= 2)
                def _():
                    pltpu.make_async_copy(
                        out_v.at[b], agg_hbm.at[pl.ds(base + (s - 2) * CH, CH)], so[b]
                    ).wait()

                for i in range(CH):
                    for c in range(NL):
                        sl = pl.ds(c * L, L)
                        acc = rows_v[b, i * KG, sl]
                        for kk in range(1, KG):
                            acc = jnp.maximum(acc, rows_v[b, i * KG + kk, sl])
                        agg = acc + a_v[b, i, sl]
                        out_v[b, i, sl] = agg
                        s_v[sl] = s_v[sl] + agg
                        ss_v[sl] = ss_v[sl] + agg * agg

                pltpu.make_async_copy(
                    out_v.at[b], agg_hbm.at[pl.ds(base + s * CH, CH)], so[b]
                ).start()
            return carry

        lax.fori_loop(0, steps // 2, pair, 0)
        for b in (0, 1):
            pltpu.make_async_copy(
                out_v.at[b], agg_hbm.at[pl.ds(base + (steps - 2 + b) * CH, CH)], so[b]
            ).wait()
        pltpu.sync_copy(s_v, s_hbm.at[wid])
        pltpu.sync_copy(ss_v, ss_hbm.at[wid])

    return k(bf_flat, idx_flat, a_flat)


def _p5(agg_ref, sc_ref, sh_ref, w2t_ref, b2_ref, outp_ref, s_ref, ss_ref):
    z = agg_ref[0] * sc_ref[...] + sh_ref[...]
    gx = 0.5 * z * (1.0 + _erf(z * 0.7071067811865476))
    o = jnp.dot(gx, w2t_ref[...], preferred_element_type=jnp.float32) + b2_ref[...]
    outp_ref[0] = o
    s_ref[0, 0] = jnp.sum(o, axis=0)
    ss_ref[0, 0] = jnp.sum(o * o, axis=0)


def _p6(outp_ref, sc_ref, sh_ref, xT_ref, out_ref):
    out_ref[0] = outp_ref[0] * sc_ref[...] + sh_ref[...] + xT_ref[0]


def _affine(s_p, ss_p, g, be, cnt):
    m = jnp.sum(s_p, axis=(0, 1)) / cnt
    v = jnp.sum(ss_p, axis=(0, 1)) / cnt - m * m
    sc = g / jnp.sqrt(v + 1e-5)
    sh = be - m * sc
    return sc[None, :], sh[None, :]


def kernel(x, W_fc1, b_fc1, g1, be1, W_gc, b_gc, g2, be2, W_fc2, b_fc2, g3, be3):
    B, C, H, Wd = x.shape
    N = H * Wd
    HID = W_gc.shape[0]
    TN = 256 if N % 256 == 0 else N
    NT = N // TN
    f32 = jnp.float32

    xT = jnp.transpose(x.reshape(B, C, N), (0, 2, 1))
    w1t = W_fc1.T
    wi, wj = W_gc[:, :C], W_gc[:, C:]
    wdT = (wi - wj).T
    HIDP = HID
    wjT = jnp.pad(wj.T, ((0, 0), (0, HIDP - HID)))
    w2t = W_fc2.T

    yT, s1p, ss1p = pl.pallas_call(
        _p1,
        grid=(B,),
        in_specs=[
            pl.BlockSpec((1, N, C), lambda b: (b, 0, 0)),
            pl.BlockSpec((C, C), lambda b: (0, 0)),
            pl.BlockSpec((1, C), lambda b: (0, 0)),
        ],
        out_specs=[
            pl.BlockSpec((1, N, C), lambda b: (b, 0, 0)),
            pl.BlockSpec((1, 1, C), lambda b: (b, 0, 0)),
            pl.BlockSpec((1, 1, C), lambda b: (b, 0, 0)),
        ],
        out_shape=[
            jax.ShapeDtypeStruct((B, N, C), f32),
            jax.ShapeDtypeStruct((B, 1, C), f32),
            jax.ShapeDtypeStruct((B, 1, C), f32),
        ],
    )(xT, w1t, b_fc1[None, :])
    sc1, sh1 = _affine(s1p, ss1p, g1, be1, B * N)

    NSPLIT = 4 if B % 4 == 0 else 1
    BH = B // NSPLIT
    agg_chunks, st2 = [], []
    for h in range(NSPLIT):
        idx_h, aT_h, bfT_h = pl.pallas_call(
            functools.partial(_p2, TN=TN),
            grid=(BH,),
            in_specs=[
                pl.BlockSpec((1, N, C), lambda b, h=h: (h * BH + b, 0, 0)),
                pl.BlockSpec((1, C), lambda b: (0, 0)),
                pl.BlockSpec((1, C), lambda b: (0, 0)),
                pl.BlockSpec((C, HID), lambda b: (0, 0)),
                pl.BlockSpec((C, HIDP), lambda b: (0, 0)),
                pl.BlockSpec((1, HID), lambda b: (0, 0)),
            ],
            out_specs=[
                pl.BlockSpec((1, N, _KPAD), lambda b: (b, 0, 0)),
                pl.BlockSpec((1, N, HID), lambda b: (b, 0, 0)),
                pl.BlockSpec((1, N, HIDP), lambda b: (b, 0, 0)),
            ],
            out_shape=[
                jax.ShapeDtypeStruct((BH, N, _KPAD), jnp.int32),
                jax.ShapeDtypeStruct((BH, N, HID), f32),
                jax.ShapeDtypeStruct((BH, N, HIDP), f32),
            ],
            scratch_shapes=[
                pltpu.VMEM((N, C), f32),
                pltpu.VMEM((N, C + 1), f32),
            ],
        )(yT, sc1, sh1, wdT, wjT, b_gc[None, :])

        idx_flat = idx_h[:, :, :_K].reshape(BH * N * _K)
        agg_h, s2p_h, ss2p_h = _sc_gather_agg(
            bfT_h.reshape(BH * N, HIDP), idx_flat, aT_h.reshape(BH * N, HID))
        agg_chunks.append(agg_h.reshape(BH, N, HID))
        st2.append((s2p_h, ss2p_h))

    agg = jnp.concatenate(agg_chunks, axis=0) if NSPLIT > 1 else agg_chunks[0]
    s2p = jnp.concatenate([s for s, _ in st2], axis=0)
    ss2p = jnp.concatenate([ss for _, ss in st2], axis=0)
    sc2, sh2 = _affine(s2p[:, None, :], ss2p[:, None, :], g2, be2, B * N)

    outp, s3p, ss3p = pl.pallas_call(
        _p5,
        grid=(B, NT),
        in_specs=[
            pl.BlockSpec((1, TN, HID), lambda b, t: (b, t, 0)),
            pl.BlockSpec((1, HID), lambda b, t: (0, 0)),
            pl.BlockSpec((1, HID), lambda b, t: (0, 0)),
            pl.BlockSpec((HID, C), lambda b, t: (0, 0)),
            pl.BlockSpec((1, C), lambda b, t: (0, 0)),
        ],
        out_specs=[
            pl.BlockSpec((1, TN, C), lambda b, t: (b, t, 0)),
            pl.BlockSpec((1, 1, C), lambda b, t: (b * NT + t, 0, 0)),
            pl.BlockSpec((1, 1, C), lambda b, t: (b * NT + t, 0, 0)),
        ],
        out_shape=[
            jax.ShapeDtypeStruct((B, N, C), f32),
            jax.ShapeDtypeStruct((B * NT, 1, C), f32),
            jax.ShapeDtypeStruct((B * NT, 1, C), f32),
        ],
    )(agg, sc2, sh2, w2t, b_fc2[None, :])
    sc3, sh3 = _affine(s3p, ss3p, g3, be3, B * N)

    outT = pl.pallas_call(
        _p6,
        grid=(B,),
        in_specs=[
            pl.BlockSpec((1, N, C), lambda b: (b, 0, 0)),
            pl.BlockSpec((1, C), lambda b: (0, 0)),
            pl.BlockSpec((1, C), lambda b: (0, 0)),
            pl.BlockSpec((1, N, C), lambda b: (b, 0, 0)),
        ],
        out_specs=pl.BlockSpec((1, N, C), lambda b: (b, 0, 0)),
        out_shape=jax.ShapeDtypeStruct((B, N, C), f32),
    )(outp, sc3, sh3, xT)

    return jnp.transpose(outT, (0, 2, 1)).reshape(B, C, H, Wd)


# speedup vs baseline: 1.1132x; 1.0443x over previous
import functools

import jax
import jax.numpy as jnp
from jax import lax
from jax.experimental import pallas as pl
from jax.experimental.pallas import tpu as pltpu
from jax.experimental.pallas import tpu_sc as plsc

_K = 9
_KPAD = 16


def _erf(x):
    a1, a2, a3, a4, a5 = 0.254829592, -0.284496736, 1.421413741, -1.453152027, 1.061405429
    p = 0.3275911
    ax = jnp.abs(x)
    t = 1.0 / (1.0 + p * ax)
    poly = ((((a5 * t + a4) * t + a3) * t + a2) * t + a1) * t
    y = 1.0 - poly * jnp.exp(-ax * ax)
    return jnp.sign(x) * y


def _p1(xT_ref, w1t_ref, b1_ref, y_ref, s_ref, ss_ref):
    xT = xT_ref[0]
    y = jnp.dot(xT, w1t_ref[...], preferred_element_type=jnp.float32) + b1_ref[...]
    y_ref[0] = y
    s_ref[0, 0] = jnp.sum(y, axis=0)
    ss_ref[0, 0] = jnp.sum(y * y, axis=0)


def _p2(y_ref, sc_ref, sh_ref, wdT_ref, wjT_ref, bgc_ref, idx_ref, aT_ref,
        bfT_ref, h_sc, laug_sc, *, TN):
    b = pl.program_id(0)
    N = y_ref.shape[1]
    NT = N // TN
    y = y_ref[0]
    h = y * sc_ref[...] + sh_ref[...]
    h_sc[...] = h
    nrm = jnp.sqrt(jnp.sum(h * h, axis=1, keepdims=True))
    xn = h / (nrm + 1e-12)
    x2 = jnp.sum(xn * xn, axis=1, keepdims=True)
    laug_sc[...] = jnp.concatenate([xn, jnp.ones((N, 1), jnp.float32)], axis=1)
    raug = jnp.concatenate([2.0 * xn, -x2], axis=1)

    cols = lax.broadcasted_iota(jnp.int32, (TN, N), 1)
    negcols = -cols.astype(jnp.float32)
    lanek = lax.broadcasted_iota(jnp.int32, (TN, _KPAD), 1)
    iota0 = lax.broadcasted_iota(jnp.int32, (TN, 1), 0)

    def tile(t, carry):
        lt = laug_sc[pl.ds(t * TN, TN), :]
        score = lax.dot_general(lt, raug, (((1,), (1,)), ((), ())),
                                preferred_element_type=jnp.float32)
        rowid = t * TN + iota0
        acc = jnp.where(lanek == 0, rowid, 0)
        vals = jnp.where(cols == rowid, -jnp.inf, score)
        for k in range(1, _K):
            rmax = jnp.max(vals, axis=1, keepdims=True)
            amf = jnp.max(jnp.where(vals == rmax, negcols, -jnp.inf), axis=1)
            am = (-amf).astype(jnp.int32)
            acc = jnp.where(lanek == k, am[:, None], acc)
            vals = jnp.where(cols == am[:, None], -jnp.inf, vals)
        idx_ref[0, pl.ds(t * TN, TN), :] = acc + b * N
        ht = h_sc[pl.ds(t * TN, TN), :]
        aT_ref[0, pl.ds(t * TN, TN), :] = lax.dot_general(
            ht, wdT_ref[...], (((1,), (0,)), ((), ())),
            preferred_element_type=jnp.float32) + bgc_ref[...]
        bfT_ref[0, pl.ds(t * TN, TN), :] = lax.dot_general(
            ht, wjT_ref[...], (((1,), (0,)), ((), ())),
            preferred_element_type=jnp.float32)
        return carry

    lax.fori_loop(0, NT, tile, 0)


def _sc_gather_agg(bf_flat, idx_flat, a_flat):
    R, DP = bf_flat.shape
    D = a_flat.shape[1]
    KG = _K
    info = plsc.get_sparse_core_info()
    NW = info.num_cores * info.num_subcores
    L = info.num_lanes
    NL = D // L
    rows_per_w = R // NW
    CH = 8
    CHK = CH * KG
    steps = rows_per_w // CH
    mesh = plsc.VectorSubcoreMesh(core_axis_name="c", subcore_axis_name="s")

    @functools.partial(
        pl.kernel,
        mesh=mesh,
        compiler_params=pltpu.CompilerParams(use_tc_tiling_on_sc=False),
        out_type=[
            jax.ShapeDtypeStruct((R, D), jnp.float32),
            jax.ShapeDtypeStruct((NW, D), jnp.float32),
            jax.ShapeDtypeStruct((NW, D), jnp.float32),
        ],
    scratch_types=[
            pltpu.VMEM((rows_per_w * KG,), jnp.int32),
            pltpu.VMEM((2, CHK, DP), jnp.float32),
            pltpu.VMEM((2, CH, D), jnp.float32),
            pltpu.VMEM((2, CH, D), jnp.float32),
            pltpu.VMEM((D,), jnp.float32),
            pltpu.VMEM((D,), jnp.float32),
            pltpu.SemaphoreType.DMA,
            pltpu.SemaphoreType.DMA,
            pltpu.SemaphoreType.DMA,
            pltpu.SemaphoreType.DMA,
            pltpu.SemaphoreType.DMA,
            pltpu.SemaphoreType.DMA,
        ],
    )
    def k(bf_hbm, idx_hbm, a_hbm, agg_hbm, s_hbm, ss_hbm,
          idx_v, rows_v, a_v, out_v, s_v, ss_v,
          sg0, sg1, sa0, sa1, so0, so1):
        wid = lax.axis_index("s") * info.num_cores + lax.axis_index("c")
        base = wid * rows_per_w
        sg = (sg0, sg1)
        sa = (sa0, sa1)
        so = (so0, so1)
        pltpu.sync_copy(idx_hbm.at[pl.ds(base * KG, rows_per_w * KG)], idx_v)
        zero = jnp.zeros((L,), jnp.float32)
        for c in range(NL):
            s_v[pl.ds(c * L, L)] = zero
            ss_v[pl.ds(c * L, L)] = zero

        def launch(s, slot):
            pltpu.make_async_copy(
                bf_hbm.at[idx_v.at[pl.ds(s * CHK, CHK)]], rows_v.at[slot], sg[slot]
            ).start()
            pltpu.make_async_copy(
                a_hbm.at[pl.ds(base + s * CH, CH)], a_v.at[slot], sa[slot]
            ).start()

        def wait_launch(s, slot):
            pltpu.make_async_copy(
                bf_hbm.at[idx_v.at[pl.ds(s * CHK, CHK)]], rows_v.at[slot], sg[slot]
            ).wait()
            pltpu.make_async_copy(
                a_hbm.at[pl.ds(base + s * CH, CH)], a_v.at[slot], sa[slot]
            ).wait()

        launch(0, 0)

        def pair(gp, carry):
            for b in (0, 1):
                s = 2 * gp + b

                @pl.when(s + 1 < steps)
                def _():
                    launch(s + 1, 1 - b)

                wait_launch(s, b)

                @pl.when(s >= 2)
                def _():
                    pltpu.make_async_copy(
                        out_v.at[b], agg_hbm.at[pl.ds(base + (s - 2) * CH, CH)], so[b]
                    ).wait()

                for i in range(CH):
                    for c in range(NL):
                        sl = pl.ds(c * L, L)
                        acc = rows_v[b, i * KG, sl]
                        for kk in range(1, KG):
                            acc = jnp.maximum(acc, rows_v[b, i * KG + kk, sl])
                        agg = acc + a_v[b, i, sl]
                        out_v[b, i, sl] = agg
                        s_v[sl] = s_v[sl] + agg
                        ss_v[sl] = ss_v[sl] + agg * agg

                pltpu.make_async_copy(
                    out_v.at[b], agg_hbm.at[pl.ds(base + s * CH, CH)], so[b]
                ).start()
            return carry

        lax.fori_loop(0, steps // 2, pair, 0)
        for b in (0, 1):
            pltpu.make_async_copy(
                out_v.at[b], agg_hbm.at[pl.ds(base + (steps - 2 + b) * CH, CH)], so[b]
            ).wait()
        pltpu.sync_copy(s_v, s_hbm.at[wid])
        pltpu.sync_copy(ss_v, ss_hbm.at[wid])

    return k(bf_flat, idx_flat, a_flat)


def _p5(agg_ref, sc_ref, sh_ref, w2t_ref, b2_ref, outp_ref, s_ref, ss_ref):
    z = agg_ref[0] * sc_ref[...] + sh_ref[...]
    gx = 0.5 * z * (1.0 + _erf(z * 0.7071067811865476))
    o = jnp.dot(gx, w2t_ref[...], preferred_element_type=jnp.float32) + b2_ref[...]
    outp_ref[0] = o
    s_ref[0, 0] = jnp.sum(o, axis=0)
    ss_ref[0, 0] = jnp.sum(o * o, axis=0)


def _p6(outp_ref, sc_ref, sh_ref, xT_ref, out_ref):
    out_ref[0] = outp_ref[0] * sc_ref[...] + sh_ref[...] + xT_ref[0]


def _affine(s_p, ss_p, g, be, cnt):
    m = jnp.sum(s_p, axis=(0, 1)) / cnt
    v = jnp.sum(ss_p, axis=(0, 1)) / cnt - m * m
    sc = g / jnp.sqrt(v + 1e-5)
    sh = be - m * sc
    return sc[None, :], sh[None, :]


def kernel(x, W_fc1, b_fc1, g1, be1, W_gc, b_gc, g2, be2, W_fc2, b_fc2, g3, be3):
    B, C, H, Wd = x.shape
    N = H * Wd
    HID = W_gc.shape[0]
    TN = 256 if N % 256 == 0 else N
    NT = N // TN
    f32 = jnp.float32

    xT = jnp.transpose(x.reshape(B, C, N), (0, 2, 1))
    w1t = W_fc1.T
    wi, wj = W_gc[:, :C], W_gc[:, C:]
    wdT = (wi - wj).T
    HIDP = HID
    wjT = jnp.pad(wj.T, ((0, 0), (0, HIDP - HID)))
    w2t = W_fc2.T

    yT, s1p, ss1p = pl.pallas_call(
        _p1,
        grid=(B,),
        in_specs=[
            pl.BlockSpec((1, N, C), lambda b: (b, 0, 0)),
            pl.BlockSpec((C, C), lambda b: (0, 0)),
            pl.BlockSpec((1, C), lambda b: (0, 0)),
        ],
        out_specs=[
            pl.BlockSpec((1, N, C), lambda b: (b, 0, 0)),
            pl.BlockSpec((1, 1, C), lambda b: (b, 0, 0)),
            pl.BlockSpec((1, 1, C), lambda b: (b, 0, 0)),
        ],
        out_shape=[
            jax.ShapeDtypeStruct((B, N, C), f32),
            jax.ShapeDtypeStruct((B, 1, C), f32),
            jax.ShapeDtypeStruct((B, 1, C), f32),
        ],
    )(xT, w1t, b_fc1[None, :])
    sc1, sh1 = _affine(s1p, ss1p, g1, be1, B * N)

    NSPLIT = 2 if B % 2 == 0 else 1
    BH = B // NSPLIT
    agg_chunks, st2 = [], []
    for h in range(NSPLIT):
        idx_h, aT_h, bfT_h = pl.pallas_call(
            functools.partial(_p2, TN=TN),
            grid=(BH,),
            in_specs=[
                pl.BlockSpec((1, N, C), lambda b, h=h: (h * BH + b, 0, 0)),
                pl.BlockSpec((1, C), lambda b: (0, 0)),
                pl.BlockSpec((1, C), lambda b: (0, 0)),
                pl.BlockSpec((C, HID), lambda b: (0, 0)),
                pl.BlockSpec((C, HIDP), lambda b: (0, 0)),
                pl.BlockSpec((1, HID), lambda b: (0, 0)),
            ],
            out_specs=[
                pl.BlockSpec((1, N, _KPAD), lambda b: (b, 0, 0)),
                pl.BlockSpec((1, N, HID), lambda b: (b, 0, 0)),
                pl.BlockSpec((1, N, HIDP), lambda b: (b, 0, 0)),
            ],
            out_shape=[
                jax.ShapeDtypeStruct((BH, N, _KPAD), jnp.int32),
                jax.ShapeDtypeStruct((BH, N, HID), f32),
                jax.ShapeDtypeStruct((BH, N, HIDP), f32),
            ],
            scratch_shapes=[
                pltpu.VMEM((N, C), f32),
                pltpu.VMEM((N, C + 1), f32),
            ],
        )(yT, sc1, sh1, wdT, wjT, b_gc[None, :])

        idx_flat = idx_h[:, :, :_K].reshape(BH * N * _K)
        agg_h, s2p_h, ss2p_h = _sc_gather_agg(
            bfT_h.reshape(BH * N, HIDP), idx_flat, aT_h.reshape(BH * N, HID))
        agg_chunks.append(agg_h.reshape(BH, N, HID))
        st2.append((s2p_h, ss2p_h))

    agg = jnp.concatenate(agg_chunks, axis=0) if NSPLIT > 1 else agg_chunks[0]
    s2p = jnp.concatenate([s for s, _ in st2], axis=0)
    ss2p = jnp.concatenate([ss for _, ss in st2], axis=0)
    sc2, sh2 = _affine(s2p[:, None, :], ss2p[:, None, :], g2, be2, B * N)

    outp, s3p, ss3p = pl.pallas_call(
        _p5,
        grid=(B, NT),
        in_specs=[
            pl.BlockSpec((1, TN, HID), lambda b, t: (b, t, 0)),
            pl.BlockSpec((1, HID), lambda b, t: (0, 0)),
            pl.BlockSpec((1, HID), lambda b, t: (0, 0)),
            pl.BlockSpec((HID, C), lambda b, t: (0, 0)),
            pl.BlockSpec((1, C), lambda b, t: (0, 0)),
        ],
        out_specs=[
            pl.BlockSpec((1, TN, C), lambda b, t: (b, t, 0)),
            pl.BlockSpec((1, 1, C), lambda b, t: (b * NT + t, 0, 0)),
            pl.BlockSpec((1, 1, C), lambda b, t: (b * NT + t, 0, 0)),
        ],
        out_shape=[
            jax.ShapeDtypeStruct((B, N, C), f32),
            jax.ShapeDtypeStruct((B * NT, 1, C), f32),
            jax.ShapeDtypeStruct((B * NT, 1, C), f32),
        ],
    )(agg, sc2, sh2, w2t, b_fc2[None, :])
    sc3, sh3 = _affine(s3p, ss3p, g3, be3, B * N)

    outT = pl.pallas_call(
        _p6,
        grid=(B,),
        in_specs=[
            pl.BlockSpec((1, N, C), lambda b: (b, 0, 0)),
            pl.BlockSpec((1, C), lambda b: (0, 0)),
            pl.BlockSpec((1, C), lambda b: (0, 0)),
            pl.BlockSpec((1, N, C), lambda b: (b, 0, 0)),
        ],
        out_specs=pl.BlockSpec((1, N, C), lambda b: (b, 0, 0)),
        out_shape=jax.ShapeDtypeStruct((B, N, C), f32),
    )(outp, sc3, sh3, xT)

    return jnp.transpose(outT, (0, 2, 1)).reshape(B, C, H, Wd)
